# baseline (speedup 1.0000x reference)
import functools

import jax
import jax.numpy as jnp
from jax import lax
from jax.experimental import pallas as pl
from jax.experimental.pallas import tpu as pltpu
from jax.experimental.pallas import tpu_sc as plsc

B = 512
UPAD = 144
R = 1024
HR = 2048

_bf16 = jnp.bfloat16
_f32 = jnp.float32


def _iota(n, m, dim):
    return jax.lax.broadcasted_iota(jnp.int32, (n, m), dim)


_NC, _NS = 2, 16
_NW = _NC * _NS


def _sc_hist_body(ut_hbm, s_hbm, zro_hbm, out_hbm, ut_v, s_v, loc_v):
    chunk = ut_v.shape[0]
    c = lax.axis_index("c")
    sid = lax.axis_index("s")
    wid = sid * _NC + c
    base = wid * chunk
    pltpu.sync_copy(ut_hbm.at[pl.ds(base, chunk)], ut_v)
    pltpu.sync_copy(s_hbm.at[pl.ds(base, chunk)], s_v)
    pltpu.sync_copy(zro_hbm, loc_v)
    ones = jnp.ones((16,), _f32)

    def step(i, _):
        su = s_v[pl.ds(i * 16, 16)]
        uu = ut_v[pl.ds(i * 16, 16)]
        idx = su * UPAD + uu
        plsc.addupdate_scatter(loc_v, [idx], ones, mask=su < B)
        return 0

    lax.fori_loop(0, chunk // 16, step, 0)
    pltpu.sync_copy(loc_v, out_hbm.at[pl.ds(wid * B * UPAD, B * UPAD)])


def _hsum_body(p_ref, h_ref):
    pid = pl.program_id(0)
    part = p_ref[...]

    @pl.when(pid == 0)
    def _():
        h_ref[...] = part

    @pl.when(pid != 0)
    def _():
        h_ref[...] += part


def _layer0_body(ut_ref, s_ref, h_ref, emb_ref, wa_ref, ba_ref, wb_ref,
                 bb_ref, out_ref, sums_ref, ztab, etab):
    pid = pl.program_id(0)

    @pl.when(pid == 0)
    def _():
        Hf = h_ref[...]
        cnt = jnp.sum(Hf, axis=1, keepdims=True)
        inv = 1.0 / jnp.maximum(cnt, 1.0)
        sums0 = jnp.dot(Hf.astype(_bf16), emb_ref[...],
                        preferred_element_type=_f32)
        mean0 = sums0 * inv
        z0 = jnp.dot(mean0.astype(_bf16), wa_ref[...],
                     preferred_element_type=_f32) + ba_ref[...]
        ztab[...] = z0.astype(_bf16)
        etab[...] = jnp.dot(emb_ref[...], wa_ref[...],
                            preferred_element_type=_f32).astype(_bf16)

    ut = ut_ref[0, 0, :]
    s = s_ref[0, 0, :]
    oh_u = (ut[:, None] == _iota(1, UPAD, 1)).astype(_bf16)
    oh_s = (s[:, None] == _iota(1, B, 1)).astype(_bf16)
    pre = (jnp.dot(oh_u, etab[...], preferred_element_type=_f32) +
           jnp.dot(oh_s, ztab[...], preferred_element_type=_f32))
    hid = jax.nn.relu(pre).astype(_bf16)
    out = jnp.dot(hid, wb_ref[...], preferred_element_type=_f32) + bb_ref[...]
    out_b = out.astype(_bf16)
    out_ref[...] = out_b

    oh_sT = (_iota(B, 1, 0) == s[None, :]).astype(_bf16)
    part = jnp.dot(oh_sT, out_b, preferred_element_type=_f32)

    @pl.when(pid == 0)
    def _():
        sums_ref[...] = part

    @pl.when(pid != 0)
    def _():
        sums_ref[...] += part


def _layer_body(last, x_ref, s_ref, h_ref, sumsin_ref, wa_ref, ba_ref,
                wb_ref, bb_ref, out_ref, sums_ref, ztab):
    pid = pl.program_id(0)

    @pl.when(pid == 0)
    def _():
        cnt = jnp.sum(h_ref[...], axis=1, keepdims=True)
        inv = 1.0 / jnp.maximum(cnt, 1.0)
        mean = sumsin_ref[...] * inv
        z = jnp.dot(mean.astype(_bf16), wa_ref[...],
                    preferred_element_type=_f32) + ba_ref[...]
        ztab[...] = z.astype(_bf16)

    s = s_ref[0, 0, :]
    oh_s = (s[:, None] == _iota(1, B, 1)).astype(_bf16)
    pre = (jnp.dot(x_ref[...], wa_ref[...], preferred_element_type=_f32) +
           jnp.dot(oh_s, ztab[...], preferred_element_type=_f32))
    hid = jax.nn.relu(pre).astype(_bf16)
    out = jnp.dot(hid, wb_ref[...], preferred_element_type=_f32) + bb_ref[...]
    out_b = out.astype(_bf16)
    out_ref[...] = out if last else out_b

    oh_sT = (_iota(B, 1, 0) == s[None, :]).astype(_bf16)
    part = jnp.dot(oh_sT, out_b, preferred_element_type=_f32)

    @pl.when(pid == 0)
    def _():
        sums_ref[...] = part

    @pl.when(pid != 0)
    def _():
        sums_ref[...] += part


def _const(shape):
    return pl.BlockSpec(shape, lambda i: tuple(0 for _ in shape))


def kernel(unit_type, node2graph, emb, W0a, b0a, W0b, b0b, W1a, b1a,
           W1b, b1b, W2a, b2a, W2b, b2b):
    n = unit_type.shape[0]
    npad = ((n + HR - 1) // HR) * HR
    g_l = npad // R
    g_h = npad // HR

    ut = jnp.pad(unit_type.astype(jnp.int32), (0, npad - n))
    sg = jnp.pad(node2graph.astype(jnp.int32), (0, npad - n),
                 constant_values=B)
    ut_l = ut.reshape(g_l, 1, R)
    sg_l = sg.reshape(g_l, 1, R)
    emb_p = jnp.pad(emb, ((0, UPAD - emb.shape[0]), (0, 0))).astype(_bf16)

    seq = pltpu.CompilerParams(dimension_semantics=("arbitrary",))
    idx_spec_l = pl.BlockSpec((1, 1, R), lambda i: (i, 0, 0))

    chunk = npad // _NW
    parts = pl.kernel(
        _sc_hist_body,
        out_type=jax.ShapeDtypeStruct((_NW * B * UPAD,), _f32),
        scratch_types=[pltpu.VMEM((chunk,), jnp.int32),
                       pltpu.VMEM((chunk,), jnp.int32),
                       pltpu.VMEM((B * UPAD,), _f32)],
        mesh=plsc.VectorSubcoreMesh(core_axis_name="c", subcore_axis_name="s"),
        compiler_params=pltpu.CompilerParams(needs_layout_passes=False),
    )(ut, sg, jnp.zeros((B * UPAD,), _f32))
    parts = parts.reshape(_NW * B, UPAD)

    H = pl.pallas_call(
        _hsum_body,
        grid=(_NW,),
        in_specs=[pl.BlockSpec((B, UPAD), lambda i: (i, 0))],
        out_specs=_const((B, UPAD)),
        out_shape=jax.ShapeDtypeStruct((B, UPAD), _f32),
        compiler_params=seq,
    )(parts)

    d0, d1 = W0a.shape[0], W0b.shape[1]
    x1, sums1 = pl.pallas_call(
        _layer0_body,
        grid=(g_l,),
        in_specs=[idx_spec_l, idx_spec_l, _const((B, UPAD)),
                  _const((UPAD, d0)), _const((d0, d0)), _const((1, d0)),
                  _const((d0, d1)), _const((1, d1))],
        out_specs=[pl.BlockSpec((R, d1), lambda i: (i, 0)),
                   _const((B, d1))],
        out_shape=[jax.ShapeDtypeStruct((npad, d1), _bf16),
                   jax.ShapeDtypeStruct((B, d1), _f32)],
        scratch_shapes=[pltpu.VMEM((B, d0), _bf16),
                        pltpu.VMEM((UPAD, d0), _bf16)],
        compiler_params=seq,
    )(ut_l, sg_l, H, emb_p, W0a.astype(_bf16), b0a.reshape(1, -1),
      W0b.astype(_bf16), b0b.reshape(1, -1))

    x = x1
    sums = sums1
    for li, (Wa, ba, Wb, bb) in enumerate(
            [(W1a, b1a, W1b, b1b), (W2a, b2a, W2b, b2b)]):
        last = li == 1
        din, dout = Wa.shape[0], Wb.shape[1]
        x, sums = pl.pallas_call(
            functools.partial(_layer_body, last),
            grid=(g_l,),
            in_specs=[pl.BlockSpec((R, din), lambda i: (i, 0)),
                      idx_spec_l, _const((B, UPAD)), _const((B, din)),
                      _const((din, din)), _const((1, din)),
                      _const((din, dout)), _const((1, dout))],
            out_specs=[pl.BlockSpec((R, dout), lambda i: (i, 0)),
                       _const((B, dout))],
            out_shape=[jax.ShapeDtypeStruct((npad, dout),
                                            _f32 if last else _bf16),
                       jax.ShapeDtypeStruct((B, dout), _f32)],
            scratch_shapes=[pltpu.VMEM((B, din), _bf16)],
            compiler_params=seq,
        )(x, sg_l, H, sums, Wa.astype(_bf16), ba.reshape(1, -1),
          Wb.astype(_bf16), bb.reshape(1, -1))

    return sums, x[:n]


# speedup vs baseline: 1.1045x; 1.1045x over previous
import functools

import jax
import jax.numpy as jnp
from jax import lax
from jax.experimental import pallas as pl
from jax.experimental.pallas import tpu as pltpu
from jax.experimental.pallas import tpu_sc as plsc

B = 512
UPAD = 144
R = 1024
HR = 2048

_bf16 = jnp.bfloat16
_f32 = jnp.float32


def _iota(n, m, dim):
    return jax.lax.broadcasted_iota(jnp.int32, (n, m), dim)


_NC, _NS = 2, 16
_NW = _NC * _NS


def _sc_hist_body(ut_hbm, s_hbm, zro_hbm, out_hbm, ut_v, s_v, loc_v):
    chunk = ut_v.shape[0]
    c = lax.axis_index("c")
    sid = lax.axis_index("s")
    wid = sid * _NC + c
    base = wid * chunk
    pltpu.sync_copy(ut_hbm.at[pl.ds(base, chunk)], ut_v)
    pltpu.sync_copy(s_hbm.at[pl.ds(base, chunk)], s_v)
    pltpu.sync_copy(zro_hbm, loc_v)
    ones = jnp.ones((16,), _f32)

    def step(i, _):
        su = s_v[pl.ds(i * 16, 16)]
        uu = ut_v[pl.ds(i * 16, 16)]
        idx = su * UPAD + uu
        plsc.addupdate_scatter(loc_v, [idx], ones, mask=su < B)
        return 0

    lax.fori_loop(0, chunk // 16, step, 0)
    pltpu.sync_copy(loc_v, out_hbm.at[pl.ds(wid * B * UPAD, B * UPAD)])


def _hsum_body(p_ref, h_ref):
    pid = pl.program_id(0)
    part = p_ref[...]

    @pl.when(pid == 0)
    def _():
        h_ref[...] = part

    @pl.when(pid != 0)
    def _():
        h_ref[...] += part


def _layer0_body(ut_ref, s_ref, h_ref, emb_ref, wa_ref, ba_ref, wb_ref,
                 bb_ref, out_ref, sums_ref, ztab, etab):
    pid = pl.program_id(0)

    @pl.when(pid == 0)
    def _():
        Hf = h_ref[...]
        cnt = jnp.sum(Hf, axis=1, keepdims=True)
        inv = 1.0 / jnp.maximum(cnt, 1.0)
        sums0 = jnp.dot(Hf.astype(_bf16), emb_ref[...],
                        preferred_element_type=_f32)
        mean0 = sums0 * inv
        z0 = jnp.dot(mean0.astype(_bf16), wa_ref[...],
                     preferred_element_type=_f32) + ba_ref[...]
        ztab[...] = z0.astype(_bf16)
        etab[...] = jnp.dot(emb_ref[...], wa_ref[...],
                            preferred_element_type=_f32).astype(_bf16)

    ut = ut_ref[0, 0, :]
    s = s_ref[0, 0, :]
    oh_u = (ut[:, None] == _iota(1, UPAD, 1)).astype(_bf16)
    oh_s = (s[:, None] == _iota(1, B, 1)).astype(_bf16)
    pre = (jnp.dot(oh_u, etab[...], preferred_element_type=_f32) +
           jnp.dot(oh_s, ztab[...], preferred_element_type=_f32))
    hid = jax.nn.relu(pre).astype(_bf16)
    out = jnp.dot(hid, wb_ref[...], preferred_element_type=_f32) + bb_ref[...]
    out_b = out.astype(_bf16)
    out_ref[...] = out_b

    oh_sT = (_iota(B, 1, 0) == s[None, :]).astype(_bf16)
    part = jnp.dot(oh_sT, out_b, preferred_element_type=_f32)

    @pl.when(pid == 0)
    def _():
        sums_ref[...] = part

    @pl.when(pid != 0)
    def _():
        sums_ref[...] += part


def _layer_body(last, x_ref, s_ref, h_ref, sumsin_ref, wa_ref, ba_ref,
                wb_ref, bb_ref, out_ref, sums_ref, ztab):
    pid = pl.program_id(0)

    @pl.when(pid == 0)
    def _():
        cnt = jnp.sum(h_ref[...], axis=1, keepdims=True)
        inv = 1.0 / jnp.maximum(cnt, 1.0)
        mean = sumsin_ref[...] * inv
        z = jnp.dot(mean.astype(_bf16), wa_ref[...],
                    preferred_element_type=_f32) + ba_ref[...]
        ztab[...] = z.astype(_bf16)

    s = s_ref[0, 0, :]
    oh_s = (s[:, None] == _iota(1, B, 1)).astype(_bf16)
    pre = (jnp.dot(x_ref[...], wa_ref[...], preferred_element_type=_f32) +
           jnp.dot(oh_s, ztab[...], preferred_element_type=_f32))
    hid = jax.nn.relu(pre).astype(_bf16)
    out = jnp.dot(hid, wb_ref[...], preferred_element_type=_f32) + bb_ref[...]
    out_b = out.astype(_bf16)
    out_ref[...] = out if last else out_b

    oh_sT = (_iota(B, 1, 0) == s[None, :]).astype(_bf16)
    part = jnp.dot(oh_sT, out_b, preferred_element_type=_f32)

    @pl.when(pid == 0)
    def _():
        sums_ref[...] = part

    @pl.when(pid != 0)
    def _():
        sums_ref[...] += part


def _const(shape):
    return pl.BlockSpec(shape, lambda i: tuple(0 for _ in shape))


def kernel(unit_type, node2graph, emb, W0a, b0a, W0b, b0b, W1a, b1a,
           W1b, b1b, W2a, b2a, W2b, b2b):
    n = unit_type.shape[0]
    npad = ((n + HR - 1) // HR) * HR
    g_l = npad // R
    g_h = npad // HR

    ut = jnp.pad(unit_type.astype(jnp.int32), (0, npad - n))
    sg = jnp.pad(node2graph.astype(jnp.int32), (0, npad - n),
                 constant_values=B)
    ut_l = ut.reshape(g_l, 1, R)
    sg_l = sg.reshape(g_l, 1, R)
    emb_p = jnp.pad(emb, ((0, UPAD - emb.shape[0]), (0, 0))).astype(_bf16)

    seq = pltpu.CompilerParams(dimension_semantics=("arbitrary",))
    idx_spec_l = pl.BlockSpec((1, 1, R), lambda i: (i, 0, 0))

    chunk = npad // _NW
    parts = pl.kernel(
        _sc_hist_body,
        out_type=jax.ShapeDtypeStruct((_NW * B * UPAD,), _f32),
        scratch_types=[pltpu.VMEM((chunk,), jnp.int32),
                       pltpu.VMEM((chunk,), jnp.int32),
                       pltpu.VMEM((B * UPAD,), _f32)],
        mesh=plsc.VectorSubcoreMesh(core_axis_name="c", subcore_axis_name="s"),
        compiler_params=pltpu.CompilerParams(needs_layout_passes=False),
    )(ut, sg, jnp.zeros((B * UPAD,), _f32))
    parts = parts.reshape(_NW * B, UPAD)

    H = pl.pallas_call(
        _hsum_body,
        grid=(_NW,),
        in_specs=[pl.BlockSpec((B, UPAD), lambda i: (i, 0))],
        out_specs=_const((B, UPAD)),
        out_shape=jax.ShapeDtypeStruct((B, UPAD), _f32),
        compiler_params=seq,
    )(parts)

    d0, d1 = W0a.shape[0], W0b.shape[1]
    x1, sums1 = pl.pallas_call(
        _layer0_body,
        grid=(g_l,),
        in_specs=[idx_spec_l, idx_spec_l, _const((B, UPAD)),
                  _const((UPAD, d0)), _const((d0, d0)), _const((1, d0)),
                  _const((d0, d1)), _const((1, d1))],
        out_specs=[pl.BlockSpec((R, d1), lambda i: (i, 0)),
                   _const((B, d1))],
        out_shape=[jax.ShapeDtypeStruct((npad, d1), _bf16),
                   jax.ShapeDtypeStruct((B, d1), _f32)],
        scratch_shapes=[pltpu.VMEM((B, d0), _bf16),
                        pltpu.VMEM((UPAD, d0), _bf16)],
        compiler_params=seq,
    )(ut_l, sg_l, H, emb_p, W0a.astype(_bf16), b0a.reshape(1, -1),
      W0b.astype(_bf16), b0b.reshape(1, -1))

    x = x1
    sums = sums1
    for li, (Wa, ba, Wb, bb) in enumerate(
            [(W1a, b1a, W1b, b1b), (W2a, b2a, W2b, b2b)]):
        last = li == 1
        din, dout = Wa.shape[0], Wb.shape[1]
        x, sums = pl.pallas_call(
            functools.partial(_layer_body, last),
            grid=(g_l,),
            in_specs=[pl.BlockSpec((R, din), lambda i: (i, 0)),
                      idx_spec_l, _const((B, UPAD)), _const((B, din)),
                      _const((din, din)), _const((1, din)),
                      _const((din, dout)), _const((1, dout))],
            out_specs=[pl.BlockSpec((R, dout), lambda i: (i, 0)),
                       _const((B, dout))],
            out_shape=[jax.ShapeDtypeStruct((n if last else npad, dout),
                                            _f32 if last else _bf16),
                       jax.ShapeDtypeStruct((B, dout), _f32)],
            scratch_shapes=[pltpu.VMEM((B, din), _bf16)],
            compiler_params=seq,
        )(x, sg_l, H, sums, Wa.astype(_bf16), ba.reshape(1, -1),
          Wb.astype(_bf16), bb.reshape(1, -1))

    return sums, x


# speedup vs baseline: 1.2322x; 1.1156x over previous
import functools

import jax
import jax.numpy as jnp
from jax import lax
from jax.experimental import pallas as pl
from jax.experimental.pallas import tpu as pltpu
from jax.experimental.pallas import tpu_sc as plsc

B = 512
UPAD = 144
R = 2048
HR = 2048

_bf16 = jnp.bfloat16
_f32 = jnp.float32


def _iota(n, m, dim):
    return jax.lax.broadcasted_iota(jnp.int32, (n, m), dim)


_NC, _NS = 2, 16
_NW = _NC * _NS


def _sc_hist_body(ut_hbm, s_hbm, zro_hbm, out_hbm, ut_v, s_v, loc_v):
    chunk = ut_v.shape[0]
    c = lax.axis_index("c")
    sid = lax.axis_index("s")
    wid = sid * _NC + c
    base = wid * chunk
    pltpu.sync_copy(ut_hbm.at[pl.ds(base, chunk)], ut_v)
    pltpu.sync_copy(s_hbm.at[pl.ds(base, chunk)], s_v)
    pltpu.sync_copy(zro_hbm, loc_v)
    ones = jnp.ones((16,), _f32)

    def step(i, _):
        su = s_v[pl.ds(i * 16, 16)]
        uu = ut_v[pl.ds(i * 16, 16)]
        idx = su * UPAD + uu
        plsc.addupdate_scatter(loc_v, [idx], ones, mask=su < B)
        return 0

    lax.fori_loop(0, chunk // 16, step, 0)
    pltpu.sync_copy(loc_v, out_hbm.at[pl.ds(wid * B * UPAD, B * UPAD)])


def _hsum_body(p_ref, h_ref):
    pid = pl.program_id(0)
    part = p_ref[...]

    @pl.when(pid == 0)
    def _():
        h_ref[...] = part

    @pl.when(pid != 0)
    def _():
        h_ref[...] += part


def _layer0_body(ut_ref, s_ref, h_ref, emb_ref, wa_ref, ba_ref, wb_ref,
                 bb_ref, out_ref, sums_ref, ztab, etab):
    pid = pl.program_id(0)

    @pl.when(pid == 0)
    def _():
        Hf = h_ref[...]
        cnt = jnp.sum(Hf, axis=1, keepdims=True)
        inv = 1.0 / jnp.maximum(cnt, 1.0)
        sums0 = jnp.dot(Hf.astype(_bf16), emb_ref[...],
                        preferred_element_type=_f32)
        mean0 = sums0 * inv
        z0 = jnp.dot(mean0.astype(_bf16), wa_ref[...],
                     preferred_element_type=_f32) + ba_ref[...]
        ztab[...] = z0.astype(_bf16)
        etab[...] = jnp.dot(emb_ref[...], wa_ref[...],
                            preferred_element_type=_f32).astype(_bf16)

    ut = ut_ref[0, 0, :]
    s = s_ref[0, 0, :]
    oh_u = (ut[:, None] == _iota(1, UPAD, 1)).astype(_bf16)
    oh_s = (s[:, None] == _iota(1, B, 1)).astype(_bf16)
    pre = (jnp.dot(oh_u, etab[...], preferred_element_type=_f32) +
           jnp.dot(oh_s, ztab[...], preferred_element_type=_f32))
    hid = jax.nn.relu(pre).astype(_bf16)
    out = jnp.dot(hid, wb_ref[...], preferred_element_type=_f32) + bb_ref[...]
    out_b = out.astype(_bf16)
    out_ref[...] = out_b

    oh_sT = (_iota(B, 1, 0) == s[None, :]).astype(_bf16)
    part = jnp.dot(oh_sT, out_b, preferred_element_type=_f32)

    @pl.when(pid == 0)
    def _():
        sums_ref[...] = part

    @pl.when(pid != 0)
    def _():
        sums_ref[...] += part


def _layer_body(last, x_ref, s_ref, h_ref, sumsin_ref, wa_ref, ba_ref,
                wb_ref, bb_ref, out_ref, sums_ref, ztab):
    pid = pl.program_id(0)

    @pl.when(pid == 0)
    def _():
        cnt = jnp.sum(h_ref[...], axis=1, keepdims=True)
        inv = 1.0 / jnp.maximum(cnt, 1.0)
        mean = sumsin_ref[...] * inv
        z = jnp.dot(mean.astype(_bf16), wa_ref[...],
                    preferred_element_type=_f32) + ba_ref[...]
        ztab[...] = z.astype(_bf16)

    s = s_ref[0, 0, :]
    oh_s = (s[:, None] == _iota(1, B, 1)).astype(_bf16)
    pre = (jnp.dot(x_ref[...], wa_ref[...], preferred_element_type=_f32) +
           jnp.dot(oh_s, ztab[...], preferred_element_type=_f32))
    hid = jax.nn.relu(pre).astype(_bf16)
    out = jnp.dot(hid, wb_ref[...], preferred_element_type=_f32) + bb_ref[...]
    out_b = out.astype(_bf16)
    out_ref[...] = out if last else out_b

    oh_sT = (_iota(B, 1, 0) == s[None, :]).astype(_bf16)
    part = jnp.dot(oh_sT, out_b, preferred_element_type=_f32)

    @pl.when(pid == 0)
    def _():
        sums_ref[...] = part

    @pl.when(pid != 0)
    def _():
        sums_ref[...] += part


def _const(shape):
    return pl.BlockSpec(shape, lambda i: tuple(0 for _ in shape))


def kernel(unit_type, node2graph, emb, W0a, b0a, W0b, b0b, W1a, b1a,
           W1b, b1b, W2a, b2a, W2b, b2b):
    n = unit_type.shape[0]
    npad = ((n + HR - 1) // HR) * HR
    g_l = npad // R
    g_h = npad // HR

    ut = jnp.pad(unit_type.astype(jnp.int32), (0, npad - n))
    sg = jnp.pad(node2graph.astype(jnp.int32), (0, npad - n),
                 constant_values=B)
    ut_l = ut.reshape(g_l, 1, R)
    sg_l = sg.reshape(g_l, 1, R)
    emb_p = jnp.pad(emb, ((0, UPAD - emb.shape[0]), (0, 0))).astype(_bf16)

    seq = pltpu.CompilerParams(dimension_semantics=("arbitrary",))
    idx_spec_l = pl.BlockSpec((1, 1, R), lambda i: (i, 0, 0))

    chunk = npad // _NW
    parts = pl.kernel(
        _sc_hist_body,
        out_type=jax.ShapeDtypeStruct((_NW * B * UPAD,), _f32),
        scratch_types=[pltpu.VMEM((chunk,), jnp.int32),
                       pltpu.VMEM((chunk,), jnp.int32),
                       pltpu.VMEM((B * UPAD,), _f32)],
        mesh=plsc.VectorSubcoreMesh(core_axis_name="c", subcore_axis_name="s"),
        compiler_params=pltpu.CompilerParams(needs_layout_passes=False),
    )(ut, sg, jnp.zeros((B * UPAD,), _f32))
    parts = parts.reshape(_NW * B, UPAD)

    H = pl.pallas_call(
        _hsum_body,
        grid=(_NW,),
        in_specs=[pl.BlockSpec((B, UPAD), lambda i: (i, 0))],
        out_specs=_const((B, UPAD)),
        out_shape=jax.ShapeDtypeStruct((B, UPAD), _f32),
        compiler_params=seq,
    )(parts)

    d0, d1 = W0a.shape[0], W0b.shape[1]
    x1, sums1 = pl.pallas_call(
        _layer0_body,
        grid=(g_l,),
        in_specs=[idx_spec_l, idx_spec_l, _const((B, UPAD)),
                  _const((UPAD, d0)), _const((d0, d0)), _const((1, d0)),
                  _const((d0, d1)), _const((1, d1))],
        out_specs=[pl.BlockSpec((R, d1), lambda i: (i, 0)),
                   _const((B, d1))],
        out_shape=[jax.ShapeDtypeStruct((npad, d1), _bf16),
                   jax.ShapeDtypeStruct((B, d1), _f32)],
        scratch_shapes=[pltpu.VMEM((B, d0), _bf16),
                        pltpu.VMEM((UPAD, d0), _bf16)],
        compiler_params=seq,
    )(ut_l, sg_l, H, emb_p, W0a.astype(_bf16), b0a.reshape(1, -1),
      W0b.astype(_bf16), b0b.reshape(1, -1))

    x = x1
    sums = sums1
    for li, (Wa, ba, Wb, bb) in enumerate(
            [(W1a, b1a, W1b, b1b), (W2a, b2a, W2b, b2b)]):
        last = li == 1
        din, dout = Wa.shape[0], Wb.shape[1]
        x, sums = pl.pallas_call(
            functools.partial(_layer_body, last),
            grid=(g_l,),
            in_specs=[pl.BlockSpec((R, din), lambda i: (i, 0)),
                      idx_spec_l, _const((B, UPAD)), _const((B, din)),
                      _const((din, din)), _const((1, din)),
                      _const((din, dout)), _const((1, dout))],
            out_specs=[pl.BlockSpec((R, dout), lambda i: (i, 0)),
                       _const((B, dout))],
            out_shape=[jax.ShapeDtypeStruct((n if last else npad, dout),
                                            _f32 if last else _bf16),
                       jax.ShapeDtypeStruct((B, dout), _f32)],
            scratch_shapes=[pltpu.VMEM((B, din), _bf16)],
            compiler_params=seq,
        )(x, sg_l, H, sums, Wa.astype(_bf16), ba.reshape(1, -1),
          Wb.astype(_bf16), bb.reshape(1, -1))

    return sums, x


# speedup vs baseline: 1.2826x; 1.0409x over previous
import functools

import jax
import jax.numpy as jnp
from jax import lax
from jax.experimental import pallas as pl
from jax.experimental.pallas import tpu as pltpu
from jax.experimental.pallas import tpu_sc as plsc

B = 512
UPAD = 144
R = 4096
HR = 4096

_bf16 = jnp.bfloat16
_f32 = jnp.float32


def _iota(n, m, dim):
    return jax.lax.broadcasted_iota(jnp.int32, (n, m), dim)


_NC, _NS = 2, 16
_NW = _NC * _NS


def _sc_hist_body(ut_hbm, s_hbm, zro_hbm, out_hbm, ut_v, s_v, loc_v):
    chunk = ut_v.shape[0]
    c = lax.axis_index("c")
    sid = lax.axis_index("s")
    wid = sid * _NC + c
    base = wid * chunk
    pltpu.sync_copy(ut_hbm.at[pl.ds(base, chunk)], ut_v)
    pltpu.sync_copy(s_hbm.at[pl.ds(base, chunk)], s_v)
    pltpu.sync_copy(zro_hbm, loc_v)
    ones = jnp.ones((16,), _f32)

    def step(i, _):
        su = s_v[pl.ds(i * 16, 16)]
        uu = ut_v[pl.ds(i * 16, 16)]
        idx = su * UPAD + uu
        plsc.addupdate_scatter(loc_v, [idx], ones, mask=su < B)
        return 0

    lax.fori_loop(0, chunk // 16, step, 0)
    pltpu.sync_copy(loc_v, out_hbm.at[pl.ds(wid * B * UPAD, B * UPAD)])


def _hsum_body(p_ref, h_ref):
    pid = pl.program_id(0)
    part = p_ref[...]

    @pl.when(pid == 0)
    def _():
        h_ref[...] = part

    @pl.when(pid != 0)
    def _():
        h_ref[...] += part


def _layer0_body(ut_ref, s_ref, h_ref, emb_ref, wa_ref, ba_ref, wb_ref,
                 bb_ref, out_ref, sums_ref, ztab, etab):
    pid = pl.program_id(0)

    @pl.when(pid == 0)
    def _():
        Hf = h_ref[...]
        cnt = jnp.sum(Hf, axis=1, keepdims=True)
        inv = 1.0 / jnp.maximum(cnt, 1.0)
        sums0 = jnp.dot(Hf.astype(_bf16), emb_ref[...],
                        preferred_element_type=_f32)
        mean0 = sums0 * inv
        z0 = jnp.dot(mean0.astype(_bf16), wa_ref[...],
                     preferred_element_type=_f32) + ba_ref[...]
        ztab[...] = z0.astype(_bf16)
        etab[...] = jnp.dot(emb_ref[...], wa_ref[...],
                            preferred_element_type=_f32).astype(_bf16)

    ut = ut_ref[0, 0, :]
    s = s_ref[0, 0, :]
    oh_u = (ut[:, None] == _iota(1, UPAD, 1)).astype(_bf16)
    oh_s = (s[:, None] == _iota(1, B, 1)).astype(_bf16)
    pre = (jnp.dot(oh_u, etab[...], preferred_element_type=_f32) +
           jnp.dot(oh_s, ztab[...], preferred_element_type=_f32))
    hid = jax.nn.relu(pre).astype(_bf16)
    out = jnp.dot(hid, wb_ref[...], preferred_element_type=_f32) + bb_ref[...]
    out_b = out.astype(_bf16)
    out_ref[...] = out_b

    oh_sT = (_iota(B, 1, 0) == s[None, :]).astype(_bf16)
    part = jnp.dot(oh_sT, out_b, preferred_element_type=_f32)

    @pl.when(pid == 0)
    def _():
        sums_ref[...] = part

    @pl.when(pid != 0)
    def _():
        sums_ref[...] += part


def _layer_body(last, x_ref, s_ref, h_ref, sumsin_ref, wa_ref, ba_ref,
                wb_ref, bb_ref, out_ref, sums_ref, ztab):
    pid = pl.program_id(0)

    @pl.when(pid == 0)
    def _():
        cnt = jnp.sum(h_ref[...], axis=1, keepdims=True)
        inv = 1.0 / jnp.maximum(cnt, 1.0)
        mean = sumsin_ref[...] * inv
        z = jnp.dot(mean.astype(_bf16), wa_ref[...],
                    preferred_element_type=_f32) + ba_ref[...]
        ztab[...] = z.astype(_bf16)

    s = s_ref[0, 0, :]
    oh_s = (s[:, None] == _iota(1, B, 1)).astype(_bf16)
    pre = (jnp.dot(x_ref[...], wa_ref[...], preferred_element_type=_f32) +
           jnp.dot(oh_s, ztab[...], preferred_element_type=_f32))
    hid = jax.nn.relu(pre).astype(_bf16)
    out = jnp.dot(hid, wb_ref[...], preferred_element_type=_f32) + bb_ref[...]
    out_b = out.astype(_bf16)
    out_ref[...] = out if last else out_b

    oh_sT = (_iota(B, 1, 0) == s[None, :]).astype(_bf16)
    part = jnp.dot(oh_sT, out_b, preferred_element_type=_f32)

    @pl.when(pid == 0)
    def _():
        sums_ref[...] = part

    @pl.when(pid != 0)
    def _():
        sums_ref[...] += part


def _const(shape):
    return pl.BlockSpec(shape, lambda i: tuple(0 for _ in shape))


def kernel(unit_type, node2graph, emb, W0a, b0a, W0b, b0b, W1a, b1a,
           W1b, b1b, W2a, b2a, W2b, b2b):
    n = unit_type.shape[0]
    npad = ((n + HR - 1) // HR) * HR
    g_l = npad // R
    g_h = npad // HR

    ut = jnp.pad(unit_type.astype(jnp.int32), (0, npad - n))
    sg = jnp.pad(node2graph.astype(jnp.int32), (0, npad - n),
                 constant_values=B)
    ut_l = ut.reshape(g_l, 1, R)
    sg_l = sg.reshape(g_l, 1, R)
    emb_p = jnp.pad(emb, ((0, UPAD - emb.shape[0]), (0, 0))).astype(_bf16)

    seq = pltpu.CompilerParams(dimension_semantics=("arbitrary",))
    idx_spec_l = pl.BlockSpec((1, 1, R), lambda i: (i, 0, 0))

    chunk = npad // _NW
    parts = pl.kernel(
        _sc_hist_body,
        out_type=jax.ShapeDtypeStruct((_NW * B * UPAD,), _f32),
        scratch_types=[pltpu.VMEM((chunk,), jnp.int32),
                       pltpu.VMEM((chunk,), jnp.int32),
                       pltpu.VMEM((B * UPAD,), _f32)],
        mesh=plsc.VectorSubcoreMesh(core_axis_name="c", subcore_axis_name="s"),
        compiler_params=pltpu.CompilerParams(needs_layout_passes=False),
    )(ut, sg, jnp.zeros((B * UPAD,), _f32))
    parts = parts.reshape(_NW * B, UPAD)

    H = pl.pallas_call(
        _hsum_body,
        grid=(_NW,),
        in_specs=[pl.BlockSpec((B, UPAD), lambda i: (i, 0))],
        out_specs=_const((B, UPAD)),
        out_shape=jax.ShapeDtypeStruct((B, UPAD), _f32),
        compiler_params=seq,
    )(parts)

    d0, d1 = W0a.shape[0], W0b.shape[1]
    x1, sums1 = pl.pallas_call(
        _layer0_body,
        grid=(g_l,),
        in_specs=[idx_spec_l, idx_spec_l, _const((B, UPAD)),
                  _const((UPAD, d0)), _const((d0, d0)), _const((1, d0)),
                  _const((d0, d1)), _const((1, d1))],
        out_specs=[pl.BlockSpec((R, d1), lambda i: (i, 0)),
                   _const((B, d1))],
        out_shape=[jax.ShapeDtypeStruct((npad, d1), _bf16),
                   jax.ShapeDtypeStruct((B, d1), _f32)],
        scratch_shapes=[pltpu.VMEM((B, d0), _bf16),
                        pltpu.VMEM((UPAD, d0), _bf16)],
        compiler_params=seq,
    )(ut_l, sg_l, H, emb_p, W0a.astype(_bf16), b0a.reshape(1, -1),
      W0b.astype(_bf16), b0b.reshape(1, -1))

    x = x1
    sums = sums1
    for li, (Wa, ba, Wb, bb) in enumerate(
            [(W1a, b1a, W1b, b1b), (W2a, b2a, W2b, b2b)]):
        last = li == 1
        din, dout = Wa.shape[0], Wb.shape[1]
        x, sums = pl.pallas_call(
            functools.partial(_layer_body, last),
            grid=(g_l,),
            in_specs=[pl.BlockSpec((R, din), lambda i: (i, 0)),
                      idx_spec_l, _const((B, UPAD)), _const((B, din)),
                      _const((din, din)), _const((1, din)),
                      _const((din, dout)), _const((1, dout))],
            out_specs=[pl.BlockSpec((R, dout), lambda i: (i, 0)),
                       _const((B, dout))],
            out_shape=[jax.ShapeDtypeStruct((n if last else npad, dout),
                                            _f32 if last else _bf16),
                       jax.ShapeDtypeStruct((B, dout), _f32)],
            scratch_shapes=[pltpu.VMEM((B, din), _bf16)],
            compiler_params=seq,
        )(x, sg_l, H, sums, Wa.astype(_bf16), ba.reshape(1, -1),
          Wb.astype(_bf16), bb.reshape(1, -1))

    return sums, x


# speedup vs baseline: 1.2837x; 1.0009x over previous
import functools

import jax
import jax.numpy as jnp
from jax import lax
from jax.experimental import pallas as pl
from jax.experimental.pallas import tpu as pltpu
from jax.experimental.pallas import tpu_sc as plsc

B = 512
UPAD = 144
R = 4096
HR = 4096

_bf16 = jnp.bfloat16
_f32 = jnp.float32


def _iota(n, m, dim):
    return jax.lax.broadcasted_iota(jnp.int32, (n, m), dim)


_NC, _NS = 2, 16
_NW = _NC * _NS


def _sc_hist_body(ut_hbm, s_hbm, zro_hbm, out_hbm, ut_v, s_v, loc_v):
    chunk = ut_v.shape[0]
    c = lax.axis_index("c")
    sid = lax.axis_index("s")
    wid = sid * _NC + c
    base = wid * chunk
    pltpu.sync_copy(ut_hbm.at[pl.ds(base, chunk)], ut_v)
    pltpu.sync_copy(s_hbm.at[pl.ds(base, chunk)], s_v)
    pltpu.sync_copy(zro_hbm, loc_v)
    ones = jnp.ones((16,), _f32)

    def step(i, _):
        su = s_v[pl.ds(i * 16, 16)]
        uu = ut_v[pl.ds(i * 16, 16)]
        idx = su * UPAD + uu
        plsc.addupdate_scatter(loc_v, [idx], ones, mask=su < B)
        return 0

    lax.fori_loop(0, chunk // 16, step, 0)
    pltpu.sync_copy(loc_v, out_hbm.at[pl.ds(wid * B * UPAD, B * UPAD)])


def _hsum_body(p_ref, h_ref):
    pid = pl.program_id(0)
    part = p_ref[...]

    @pl.when(pid == 0)
    def _():
        h_ref[...] = part

    @pl.when(pid != 0)
    def _():
        h_ref[...] += part


def _layer0_body(ut_ref, s_ref, h_ref, emb_ref, wa_ref, ba_ref, wb_ref,
                 bb_ref, out_ref, sums_ref, ztab, etab):
    pid = pl.program_id(0)

    @pl.when(pid == 0)
    def _():
        Hf = h_ref[...]
        cnt = jnp.sum(Hf, axis=1, keepdims=True)
        inv = 1.0 / jnp.maximum(cnt, 1.0)
        sums0 = jnp.dot(Hf.astype(_bf16), emb_ref[...],
                        preferred_element_type=_f32)
        mean0 = sums0 * inv
        z0 = jnp.dot(mean0.astype(_bf16), wa_ref[...],
                     preferred_element_type=_f32) + ba_ref[...]
        ztab[...] = z0.astype(_bf16)
        etab[...] = jnp.dot(emb_ref[...], wa_ref[...],
                            preferred_element_type=_f32).astype(_bf16)

    ut = ut_ref[0, 0, :]
    s = s_ref[0, 0, :]
    oh_u = (ut[:, None] == _iota(1, UPAD, 1)).astype(_bf16)
    oh_s = (s[:, None] == _iota(1, B, 1)).astype(_bf16)
    pre = (jnp.dot(oh_u, etab[...], preferred_element_type=_f32) +
           jnp.dot(oh_s, ztab[...], preferred_element_type=_f32))
    hid = jax.nn.relu(pre).astype(_bf16)
    out = jnp.dot(hid, wb_ref[...], preferred_element_type=_f32) + bb_ref[...]
    out_b = out.astype(_bf16)
    out_ref[...] = out_b

    oh_sT = (_iota(B, 1, 0) == s[None, :]).astype(_bf16)
    part = jnp.dot(oh_sT, hid, preferred_element_type=_f32)

    @pl.when(pid == 0)
    def _():
        sums_ref[...] = part

    @pl.when(pid != 0)
    def _():
        sums_ref[...] += part


def _layer_body(last, first, x_ref, s_ref, h_ref, sumsin_ref, pwb_ref,
                pbb_ref, wa_ref, ba_ref, wb_ref, bb_ref, out_ref, sums_ref,
                ztab):
    pid = pl.program_id(0)

    @pl.when(pid == 0)
    def _():
        cnt = jnp.sum(h_ref[...], axis=1, keepdims=True)
        inv = 1.0 / jnp.maximum(cnt, 1.0)
        if first:
            sums = (jnp.dot(sumsin_ref[...].astype(_bf16), pwb_ref[...],
                            preferred_element_type=_f32)
                    + cnt * pbb_ref[...])
        else:
            sums = sumsin_ref[...]
        mean = sums * inv
        z = jnp.dot(mean.astype(_bf16), wa_ref[...],
                    preferred_element_type=_f32) + ba_ref[...]
        ztab[...] = z.astype(_bf16)

    s = s_ref[0, 0, :]
    oh_s = (s[:, None] == _iota(1, B, 1)).astype(_bf16)
    pre = (jnp.dot(x_ref[...], wa_ref[...], preferred_element_type=_f32) +
           jnp.dot(oh_s, ztab[...], preferred_element_type=_f32))
    hid = jax.nn.relu(pre).astype(_bf16)
    out = jnp.dot(hid, wb_ref[...], preferred_element_type=_f32) + bb_ref[...]
    out_b = out.astype(_bf16)
    out_ref[...] = out if last else out_b

    oh_sT = (_iota(B, 1, 0) == s[None, :]).astype(_bf16)
    part = jnp.dot(oh_sT, out_b, preferred_element_type=_f32)

    @pl.when(pid == 0)
    def _():
        sums_ref[...] = part

    @pl.when(pid != 0)
    def _():
        sums_ref[...] += part


def _const(shape):
    return pl.BlockSpec(shape, lambda i: tuple(0 for _ in shape))


def kernel(unit_type, node2graph, emb, W0a, b0a, W0b, b0b, W1a, b1a,
           W1b, b1b, W2a, b2a, W2b, b2b):
    n = unit_type.shape[0]
    npad = ((n + HR - 1) // HR) * HR
    g_l = npad // R
    g_h = npad // HR

    ut = jnp.pad(unit_type.astype(jnp.int32), (0, npad - n))
    sg = jnp.pad(node2graph.astype(jnp.int32), (0, npad - n),
                 constant_values=B)
    ut_l = ut.reshape(g_l, 1, R)
    sg_l = sg.reshape(g_l, 1, R)
    emb_p = jnp.pad(emb, ((0, UPAD - emb.shape[0]), (0, 0))).astype(_bf16)

    seq = pltpu.CompilerParams(dimension_semantics=("arbitrary",))
    idx_spec_l = pl.BlockSpec((1, 1, R), lambda i: (i, 0, 0))

    chunk = npad // _NW
    parts = pl.kernel(
        _sc_hist_body,
        out_type=jax.ShapeDtypeStruct((_NW * B * UPAD,), _f32),
        scratch_types=[pltpu.VMEM((chunk,), jnp.int32),
                       pltpu.VMEM((chunk,), jnp.int32),
                       pltpu.VMEM((B * UPAD,), _f32)],
        mesh=plsc.VectorSubcoreMesh(core_axis_name="c", subcore_axis_name="s"),
        compiler_params=pltpu.CompilerParams(needs_layout_passes=False),
    )(ut, sg, jnp.zeros((B * UPAD,), _f32))
    parts = parts.reshape(_NW * B, UPAD)

    H = pl.pallas_call(
        _hsum_body,
        grid=(_NW,),
        in_specs=[pl.BlockSpec((B, UPAD), lambda i: (i, 0))],
        out_specs=_const((B, UPAD)),
        out_shape=jax.ShapeDtypeStruct((B, UPAD), _f32),
        compiler_params=seq,
    )(parts)

    d0, d1 = W0a.shape[0], W0b.shape[1]
    x1, sums1 = pl.pallas_call(
        _layer0_body,
        grid=(g_l,),
        in_specs=[idx_spec_l, idx_spec_l, _const((B, UPAD)),
                  _const((UPAD, d0)), _const((d0, d0)), _const((1, d0)),
                  _const((d0, d1)), _const((1, d1))],
        out_specs=[pl.BlockSpec((R, d1), lambda i: (i, 0)),
                   _const((B, d0))],
        out_shape=[jax.ShapeDtypeStruct((npad, d1), _bf16),
                   jax.ShapeDtypeStruct((B, d0), _f32)],
        scratch_shapes=[pltpu.VMEM((B, d0), _bf16),
                        pltpu.VMEM((UPAD, d0), _bf16)],
        compiler_params=seq,
    )(ut_l, sg_l, H, emb_p, W0a.astype(_bf16), b0a.reshape(1, -1),
      W0b.astype(_bf16), b0b.reshape(1, -1))

    x = x1
    sums = sums1
    for li, (Wa, ba, Wb, bb) in enumerate(
            [(W1a, b1a, W1b, b1b), (W2a, b2a, W2b, b2b)]):
        first = li == 0
        last = li == 1
        din, dout = Wa.shape[0], Wb.shape[1]
        sin_w = d0 if first else din
        x, sums = pl.pallas_call(
            functools.partial(_layer_body, last, first),
            grid=(g_l,),
            in_specs=[pl.BlockSpec((R, din), lambda i: (i, 0)),
                      idx_spec_l, _const((B, UPAD)), _const((B, sin_w)),
                      _const((sin_w, din)), _const((1, din)),
                      _const((din, din)), _const((1, din)),
                      _const((din, dout)), _const((1, dout))],
            out_specs=[pl.BlockSpec((R, dout), lambda i: (i, 0)),
                       _const((B, dout))],
            out_shape=[jax.ShapeDtypeStruct((n if last else npad, dout),
                                            _f32 if last else _bf16),
                       jax.ShapeDtypeStruct((B, dout), _f32)],
            scratch_shapes=[pltpu.VMEM((B, din), _bf16)],
            compiler_params=seq,
        )(x, sg_l, H, sums,
          (W0b if first else Wa).astype(_bf16),
          (b0b if first else ba).reshape(1, -1),
          Wa.astype(_bf16), ba.reshape(1, -1),
          Wb.astype(_bf16), bb.reshape(1, -1))

    return sums, x
